# baseline (speedup 1.0000x reference)
import functools

import jax
import jax.numpy as jnp
from jax import lax
from jax.experimental import pallas as pl
from jax.experimental.pallas import tpu as pltpu
from jax.experimental.pallas import tpu_sc as plsc

N = 100000
D = 256
L = 16
R = 800
NCHUNK = N // R
NW = 32
KMAX = -(-NCHUNK // NW)

_mesh = plsc.VectorSubcoreMesh(core_axis_name="c", subcore_axis_name="s")

_scratch = (
    [pltpu.VMEM((L,), jnp.float32)]
    + [pltpu.VMEM((R,), jnp.int32) for _ in range(KMAX)]
    + [pltpu.VMEM((R,), jnp.float32) for _ in range(KMAX)]
    + [pltpu.SemaphoreType.DMA for _ in range(2 * KMAX)]
)


@functools.partial(
    pl.kernel,
    out_type=jax.ShapeDtypeStruct((N,), jnp.float32),
    mesh=_mesh,
    compiler_params=pltpu.CompilerParams(needs_layout_passes=False),
    scratch_types=_scratch,
)
def _sc_norm_factor(types_hbm, nc_hbm, outnf_hbm, nc_v, *scr):
    tv = scr[0:KMAX]
    nfv = scr[KMAX:2 * KMAX]
    in_sem = scr[2 * KMAX:3 * KMAX]
    out_sem = scr[3 * KMAX:4 * KMAX]

    wid = lax.axis_index("c") * 16 + lax.axis_index("s")
    pltpu.sync_copy(nc_hbm, nc_v)

    for k in range(KMAX):
        c = wid + k * NW

        @pl.when(c < NCHUNK)
        def _(k=k, c=c):
            pltpu.async_copy(types_hbm.at[pl.ds(c * R, R)], tv[k], in_sem[k])

    for k in range(KMAX):
        c = wid + k * NW

        @pl.when(c < NCHUNK)
        def _(k=k, c=c):
            pltpu.make_async_copy(types_hbm.at[pl.ds(c * R, R)], tv[k],
                                  in_sem[k]).wait()
            for j in range(R // L):
                t16 = tv[k][pl.ds(j * L, L)]
                nfv[k][pl.ds(j * L, L)] = plsc.load_gather(nc_v, [t16])
            pltpu.async_copy(nfv[k], outnf_hbm.at[pl.ds(c * R, R)],
                             out_sem[k])

    for k in range(KMAX):
        c = wid + k * NW

        @pl.when(c < NCHUNK)
        def _(k=k, c=c):
            pltpu.make_async_copy(nfv[k], outnf_hbm.at[pl.ds(c * R, R)],
                                  out_sem[k]).wait()


BM = 2000


def _tc_scale_body(nc_sref, t_ref, f_ref, o_ref):
    t = t_ref[...]
    w0 = nc_sref[0, 0]
    w1 = nc_sref[1, 0]
    w2 = nc_sref[2, 0]
    w3 = nc_sref[3, 0]
    nf = jnp.where(t == 0, w0, jnp.where(t == 1, w1, jnp.where(t == 2, w2, w3)))
    o_ref[...] = f_ref[...] * nf


_tc_scale = pl.pallas_call(
    _tc_scale_body,
    grid=(N // BM,),
    in_specs=[
        pl.BlockSpec(memory_space=pltpu.SMEM),
        pl.BlockSpec((BM, 1), lambda i: (i, 0)),
        pl.BlockSpec((BM, D), lambda i: (i, 0)),
    ],
    out_specs=pl.BlockSpec((BM, D), lambda i: (i, 0)),
    out_shape=jax.ShapeDtypeStruct((N, D), jnp.float32),
    compiler_params=pltpu.CompilerParams(
        dimension_semantics=("arbitrary",)),
)


def kernel(node_features, atom_types, norm_const):
    types = atom_types.astype(jnp.int32)
    nc_flat = jnp.pad(norm_const.reshape(-1).astype(jnp.float32),
                      (0, L - norm_const.shape[0]))
    out_nf = _sc_norm_factor(types, nc_flat)
    out_feat = _tc_scale(norm_const, types.reshape(N, 1), node_features)
    return out_feat, out_nf.reshape(N, 1)
